# baseline (speedup 1.0000x reference)
import functools
import math

import jax
import jax.numpy as jnp
from jax import lax
from jax.experimental import pallas as pl
from jax.experimental.pallas import tpu as pltpu
from jax.experimental.pallas import tpu_sc as plsc

_RATE = 4


def _scalar_at(idx_v, j):
    lanes = lax.broadcasted_iota(jnp.int32, (16,), 0)
    vec = idx_v[pl.ds((j // 16) * 16, 16)]
    sel = jnp.where(lanes == (j % 16), vec, jnp.full((16,), -(2**31), jnp.int32))
    return lax.reduce_max(sel, (0,))


@functools.lru_cache(maxsize=None)
def _make_gather(n_rows, n_out, row_elems):
    info = plsc.get_sparse_core_info()
    nc, ns = info.num_cores, info.num_subcores
    nw = nc * ns
    assert n_out % nw == 0
    b_per_w = n_out // nw

    mesh = plsc.VectorSubcoreMesh(core_axis_name="c", subcore_axis_name="s")

    @functools.partial(
        pl.kernel,
        mesh=mesh,
        out_type=jax.ShapeDtypeStruct((n_out, row_elems), jnp.float32),
        scratch_types=[
            pltpu.VMEM((b_per_w,), jnp.int32),
            pltpu.VMEM((b_per_w, row_elems), jnp.float32),
            pltpu.SemaphoreType.DMA,
        ],
        compiler_params=pltpu.CompilerParams(
            use_tc_tiling_on_sc=False, needs_layout_passes=False),
    )
    def gather_kernel(table_hbm, idx_hbm, out_hbm, idx_v, rows_v, sem):
        wid = lax.axis_index("s") * nc + lax.axis_index("c")
        base = wid * b_per_w
        pltpu.sync_copy(idx_hbm.at[pl.ds(base, b_per_w)], idx_v)
        copies = []
        for j in range(b_per_w):
            r = _scalar_at(idx_v, j)
            copies.append(pltpu.make_async_copy(
                table_hbm.at[pl.ds(r, 1)], rows_v.at[pl.ds(j, 1)], sem))
        for c in copies:
            c.start()
        for c in copies:
            c.wait()
        pltpu.sync_copy(rows_v, out_hbm.at[pl.ds(base, b_per_w)])

    return gather_kernel


def kernel(pose):
    frames, people, keypoints, dimensions = pose.shape
    rate = _RATE
    n_chunks = math.ceil(frames / rate)
    chunk_starts = jnp.arange(0, frames, rate, dtype=jnp.int32)
    random_indices = jax.random.randint(
        jax.random.key(42), (n_chunks,), 0, rate, dtype=jnp.int32)
    random_indices = random_indices + chunk_starts
    random_indices = random_indices.at[-1].set(
        jnp.minimum(random_indices[-1], frames - 1))

    row_elems = people * keypoints * dimensions
    table = pose.reshape(frames, row_elems)
    out = _make_gather(frames, n_chunks, row_elems)(table, random_indices)
    return out.reshape(n_chunks, people, keypoints, dimensions)


# speedup vs baseline: 1.9327x; 1.9327x over previous
import functools
import math

import jax
import jax.numpy as jnp
from jax import lax
from jax.experimental import pallas as pl
from jax.experimental.pallas import tpu as pltpu
from jax.experimental.pallas import tpu_sc as plsc

_RATE = 4


@functools.lru_cache(maxsize=None)
def _make_row_gather(n_rows, n_frames, n_out):
    info = plsc.get_sparse_core_info()
    nc, ns = info.num_cores, info.num_subcores
    nw = nc * ns
    rows_per_w = math.ceil(n_rows / nw)

    mesh = plsc.VectorSubcoreMesh(core_axis_name="c", subcore_axis_name="s")

    @functools.partial(
        pl.kernel,
        mesh=mesh,
        out_type=jax.ShapeDtypeStruct((n_rows * n_out,), jnp.float32),
        scratch_types=[
            pltpu.VMEM((n_out,), jnp.int32),
            pltpu.VMEM((n_frames,), jnp.float32),
            pltpu.VMEM((n_out,), jnp.float32),
            pltpu.SemaphoreType.DMA,
        ],
        compiler_params=pltpu.CompilerParams(
            use_tc_tiling_on_sc=False, needs_layout_passes=False),
    )
    def row_gather(xt_hbm, idx_hbm, yt_hbm, idx_v, row_v, out_v, sem):
        wid = lax.axis_index("s") * nc + lax.axis_index("c")
        base = wid * rows_per_w
        pltpu.sync_copy(idx_hbm, idx_v)

        def row_body(rloc, _):
            row = base + rloc

            @pl.when(row < n_rows)
            def _():
                pltpu.sync_copy(xt_hbm.at[pl.ds(row * n_frames, n_frames)],
                                row_v)

                def j_body(j, _2):
                    iv = idx_v[pl.ds(j * 16, 16)]
                    out_v[pl.ds(j * 16, 16)] = plsc.load_gather(row_v, [iv])
                    return 0

                lax.fori_loop(0, n_out // 16, j_body, 0)
                pltpu.sync_copy(out_v, yt_hbm.at[pl.ds(row * n_out, n_out)])

            return 0

        lax.fori_loop(0, rows_per_w, row_body, 0)

    return row_gather


def kernel(pose):
    frames, people, keypoints, dimensions = pose.shape
    rate = _RATE
    n_chunks = math.ceil(frames / rate)
    chunk_starts = jnp.arange(0, frames, rate, dtype=jnp.int32)
    random_indices = jax.random.randint(
        jax.random.key(42), (n_chunks,), 0, rate, dtype=jnp.int32)
    random_indices = random_indices + chunk_starts
    random_indices = random_indices.at[-1].set(
        jnp.minimum(random_indices[-1], frames - 1))

    n_rows = keypoints * dimensions * people
    xt = pose.transpose(2, 3, 1, 0).reshape(-1)
    yt = _make_row_gather(n_rows, frames, n_chunks)(xt, random_indices)
    return yt.reshape(keypoints, dimensions, people, n_chunks).transpose(3, 2, 0, 1)


# speedup vs baseline: 3.1277x; 1.6183x over previous
import functools
import math

import jax
import jax.numpy as jnp
from jax import lax
from jax.experimental import pallas as pl
from jax.experimental.pallas import tpu as pltpu
from jax.experimental.pallas import tpu_sc as plsc

_RATE = 4
_LANE = 128


@functools.lru_cache(maxsize=None)
def _make_row_gather(n_super, n_ftiles, n_otiles, n_people):
    info = plsc.get_sparse_core_info()
    nc, ns = info.num_cores, info.num_subcores
    nw = nc * ns
    srw = math.ceil(n_super / nw)
    sr_in = n_ftiles * n_people * _LANE
    sr_out = n_otiles * n_people * _LANE
    n_out = n_otiles * _LANE

    mesh = plsc.VectorSubcoreMesh(core_axis_name="c", subcore_axis_name="s")

    @functools.partial(
        pl.kernel,
        mesh=mesh,
        out_type=jax.ShapeDtypeStruct((n_super * sr_out,), jnp.float32),
        scratch_types=[
            pltpu.VMEM((n_out,), jnp.int32),
            pltpu.VMEM((sr_in,), jnp.float32),
            pltpu.VMEM((sr_out,), jnp.float32),
            pltpu.SemaphoreType.DMA,
        ],
        compiler_params=pltpu.CompilerParams(
            use_tc_tiling_on_sc=False, needs_layout_passes=False),
    )
    def row_gather(xt_hbm, idx_hbm, yt_hbm, addr_v, row_v, out_v, sem):
        wid = lax.axis_index("s") * nc + lax.axis_index("c")
        base = wid * srw
        pltpu.sync_copy(idx_hbm, addr_v)

        def a_body(j, _):
            iv = addr_v[pl.ds(j * 16, 16)]
            addr_v[pl.ds(j * 16, 16)] = iv + lax.shift_right_logical(iv, 7) * 128
            return 0

        lax.fori_loop(0, n_out // 16, a_body, 0)

        def row_body(rloc, _):
            row = base + rloc

            @pl.when(row < n_super)
            def _():
                pltpu.sync_copy(xt_hbm.at[pl.ds(row * sr_in, sr_in)], row_v)

                def j_body(j, _2):
                    iv = addr_v[pl.ds(j * 16, 16)]
                    o = (j // 8) * 256 + (j % 8) * 16
                    out_v[pl.ds(o, 16)] = plsc.load_gather(row_v, [iv])
                    out_v[pl.ds(o + 128, 16)] = plsc.load_gather(row_v, [iv + 128])
                    return 0

                lax.fori_loop(0, n_out // 16, j_body, 0)
                pltpu.sync_copy(out_v, yt_hbm.at[pl.ds(row * sr_out, sr_out)])

            return 0

        lax.fori_loop(0, srw, row_body, 0)

    return row_gather


def kernel(pose):
    frames, people, keypoints, dimensions = pose.shape
    rate = _RATE
    n_chunks = math.ceil(frames / rate)
    chunk_starts = jnp.arange(0, frames, rate, dtype=jnp.int32)
    random_indices = jax.random.randint(
        jax.random.key(42), (n_chunks,), 0, rate, dtype=jnp.int32)
    random_indices = random_indices + chunk_starts
    random_indices = random_indices.at[-1].set(
        jnp.minimum(random_indices[-1], frames - 1))

    n_super = keypoints * dimensions
    n_ftiles = frames // _LANE
    n_otiles = n_chunks // _LANE
    xt = (pose.transpose(2, 3, 1, 0)
          .reshape(keypoints, dimensions, people, n_ftiles, _LANE)
          .transpose(0, 1, 3, 2, 4).reshape(-1))
    yt = _make_row_gather(n_super, n_ftiles, n_otiles, people)(
        xt, random_indices)
    return (yt.reshape(keypoints, dimensions, n_otiles, people, _LANE)
            .transpose(2, 4, 3, 0, 1)
            .reshape(n_chunks, people, keypoints, dimensions))


# speedup vs baseline: 4.3693x; 1.3970x over previous
import functools
import math

import jax
import jax.numpy as jnp
from jax import lax
from jax.experimental import pallas as pl
from jax.experimental.pallas import tpu as pltpu
from jax.experimental.pallas import tpu_sc as plsc

_RATE = 4
_LANE = 128
_UNROLL = 8


@functools.lru_cache(maxsize=None)
def _make_row_gather(n_super, n_ftiles, n_otiles, n_people):
    info = plsc.get_sparse_core_info()
    nc, ns = info.num_cores, info.num_subcores
    nw = nc * ns
    srw = math.ceil(n_super / nw)
    sr_in = n_ftiles * n_people * _LANE
    sr_out = n_otiles * n_people * _LANE
    n_out = n_otiles * _LANE
    n_groups = n_out // 16

    mesh = plsc.VectorSubcoreMesh(core_axis_name="c", subcore_axis_name="s")

    @functools.partial(
        pl.kernel,
        mesh=mesh,
        out_type=jax.ShapeDtypeStruct((n_super * sr_out,), jnp.float32),
        scratch_types=[
            pltpu.VMEM((n_out,), jnp.int32),
            pltpu.VMEM((sr_in,), jnp.float32),
            pltpu.VMEM((sr_in,), jnp.float32),
            pltpu.VMEM((sr_out,), jnp.float32),
            pltpu.VMEM((sr_out,), jnp.float32),
            pltpu.SemaphoreType.DMA,
            pltpu.SemaphoreType.DMA,
            pltpu.SemaphoreType.DMA,
            pltpu.SemaphoreType.DMA,
        ],
        compiler_params=pltpu.CompilerParams(
            use_tc_tiling_on_sc=False, needs_layout_passes=False),
    )
    def row_gather(xt_hbm, idx_hbm, yt_hbm, addr_v, row_a, row_b, out_a,
                   out_b, sem_a, sem_b, osem_a, osem_b):
        wid = lax.axis_index("s") * nc + lax.axis_index("c")
        base = wid * srw
        pltpu.sync_copy(idx_hbm, addr_v)

        def a_body(j, _):
            iv = addr_v[pl.ds(j * 16, 16)]
            addr_v[pl.ds(j * 16, 16)] = iv + lax.shift_right_logical(iv, 7) * 128
            return 0

        lax.fori_loop(0, n_out // 16, a_body, 0)

        rows = [(row_a, sem_a), (row_b, sem_b)]
        outs = [(out_a, osem_a), (out_b, osem_b)]

        def start_in(rloc, buf, sem):
            row = base + rloc
            @pl.when(row < n_super)
            def _():
                pltpu.make_async_copy(
                    xt_hbm.at[pl.ds(row * sr_in, sr_in)], buf, sem).start()

        start_in(0, row_a, sem_a)
        for rloc in range(srw):
            row = base + rloc
            buf, sem = rows[rloc % 2]
            obuf, osem = outs[rloc % 2]

            @pl.when(row < n_super)
            def _():
                pltpu.make_async_copy(
                    xt_hbm.at[pl.ds(row * sr_in, sr_in)], buf, sem).wait()

            if rloc + 1 < srw:
                nbuf, nsem = rows[(rloc + 1) % 2]
                start_in(rloc + 1, nbuf, nsem)

            @pl.when(row < n_super)
            def _():
                if rloc >= 2:
                    pltpu.make_async_copy(
                        obuf, yt_hbm.at[pl.ds((row - 2) * sr_out, sr_out)],
                        osem).wait()

                def j_body(g, _2):
                    for u in range(_UNROLL):
                        j = g * _UNROLL + u
                        iv = addr_v[pl.ds(j * 16, 16)]
                        o = (j // 8) * 256 + (j % 8) * 16
                        obuf[pl.ds(o, 16)] = plsc.load_gather(buf, [iv])
                        obuf[pl.ds(o + 128, 16)] = plsc.load_gather(
                            buf, [iv + 128])
                    return 0

                lax.fori_loop(0, n_groups // _UNROLL, j_body, 0)
                pltpu.make_async_copy(
                    obuf, yt_hbm.at[pl.ds(row * sr_out, sr_out)], osem).start()

        for rloc in range(max(srw - 2, 0), srw):
            row = base + rloc
            obuf, osem = outs[rloc % 2]

            @pl.when(row < n_super)
            def _():
                pltpu.make_async_copy(
                    obuf, yt_hbm.at[pl.ds(row * sr_out, sr_out)], osem).wait()

    return row_gather


def kernel(pose):
    frames, people, keypoints, dimensions = pose.shape
    rate = _RATE
    n_chunks = math.ceil(frames / rate)
    chunk_starts = jnp.arange(0, frames, rate, dtype=jnp.int32)
    random_indices = jax.random.randint(
        jax.random.key(42), (n_chunks,), 0, rate, dtype=jnp.int32)
    random_indices = random_indices + chunk_starts
    random_indices = random_indices.at[-1].set(
        jnp.minimum(random_indices[-1], frames - 1))

    n_super = keypoints * dimensions
    n_ftiles = frames // _LANE
    n_otiles = n_chunks // _LANE
    xt = (pose.transpose(2, 3, 1, 0)
          .reshape(keypoints, dimensions, people, n_ftiles, _LANE)
          .transpose(0, 1, 3, 2, 4).reshape(-1))
    yt = _make_row_gather(n_super, n_ftiles, n_otiles, people)(
        xt, random_indices)
    return (yt.reshape(keypoints, dimensions, n_otiles, people, _LANE)
            .transpose(2, 4, 3, 0, 1)
            .reshape(n_chunks, people, keypoints, dimensions))
